# Initial kernel scaffold; baseline (speedup 1.0000x reference)
import jax
import jax.numpy as jnp
from jax.experimental import pallas as pl


def kernel(x, edge_index, batch, params):
    raise NotImplementedError("write your pallas kernel here")


# speedup vs baseline: 1.4603x; 1.4603x over previous
import functools
import jax
import jax.numpy as jnp
import numpy as np
from jax.experimental import pallas as pl
from jax.experimental.pallas import tpu as pltpu

N_NODES = 10000
NUM_GRAPHS = 256


def _mm_body(a_ref, w_ref, b_ref, o_ref, *, slope):
    acc = jnp.dot(a_ref[...], w_ref[...], preferred_element_type=jnp.float32)
    acc = acc + b_ref[...]
    o_ref[...] = jnp.where(acc >= 0, acc, slope * acc)


def _mm(a, w, b, slope, BM=512, BN=512):
    M, K = a.shape
    _, Nc = w.shape
    Mp = ((M + BM - 1) // BM) * BM
    if Mp != M:
        a = jnp.pad(a, ((0, Mp - M), (0, 0)))
    BNe = min(BN, Nc)
    grid = (Nc // BNe, Mp // BM)
    out = pl.pallas_call(
        functools.partial(_mm_body, slope=slope),
        grid=grid,
        in_specs=[
            pl.BlockSpec((BM, K), lambda j, i: (i, 0)),
            pl.BlockSpec((K, BNe), lambda j, i: (0, j)),
            pl.BlockSpec((1, BNe), lambda j, i: (0, j)),
        ],
        out_specs=pl.BlockSpec((BM, BNe), lambda j, i: (i, j)),
        out_shape=jax.ShapeDtypeStruct((Mp, Nc), jnp.float32),
    )(a, w, b.reshape(1, -1))
    return out[:M] if Mp != M else out


def _edge_softmax_agg(h, s, d, src, dst):
    alpha = jax.nn.leaky_relu(s[src] + d[dst], 0.2)
    m = jax.ops.segment_max(alpha, dst, num_segments=N_NODES)
    ea = jnp.exp(alpha - m[dst])
    den = jax.ops.segment_sum(ea, dst, num_segments=N_NODES)
    coef = ea / (den[dst] + 1e-16)
    return jax.ops.segment_sum(coef[:, None] * h[src], dst, num_segments=N_NODES)


def kernel(x, edge_index, batch, params):
    p = params
    src = edge_index[0]
    dst = edge_index[1]
    eps = 1e-8

    cy = x[:, 0].astype(jnp.int32)
    cx = x[:, 1].astype(jnp.int32)
    ey = p['coord_emb_y'][cy]
    ex = p['coord_emb_x'][cx]
    coords = jax.nn.leaky_relu(
        jnp.concatenate([ey, ex], 1) @ p['smoosh_W'] + p['smoosh_b'], 0.01)
    pos = x[:, 2].astype(jnp.int32)
    positions = jax.nn.leaky_relu(p['pos_emb'][pos], 0.01)
    ht = x[:, 3:8]
    hold = (ht @ p['hold_W'] + p['hold_b']) / (ht.sum(1, keepdims=True) + eps)
    orr = x[:, 8:16]
    orient = (orr @ p['orient_W'] + p['orient_b']) / (orr.sum(1, keepdims=True) + eps)
    feat = jnp.concatenate([coords, orient, hold, positions], 1)

    def layer(h, Wname, asn, adn, bn):
        W = p[Wname]
        s = h @ (W @ p[asn])
        d = h @ (W @ p[adn])
        agg = _edge_softmax_agg(h, s, d, src, dst)
        return _mm(agg, W, p[bn], 0.01)

    x1 = layer(feat, 'g1_W', 'g1_as', 'g1_ad', 'g1_b')
    x2 = layer(x1, 'g2_W', 'g2_as', 'g2_ad', 'g2_b')
    x3 = layer(x2, 'g3_W', 'g3_as', 'g3_ad', 'g3_b')

    P3 = jax.ops.segment_sum(x3, batch, num_segments=NUM_GRAPHS)
    P2 = jax.ops.segment_sum(x2, batch, num_segments=NUM_GRAPHS)
    counts = jax.ops.segment_sum(jnp.ones((N_NODES,), jnp.float32), batch,
                                 num_segments=NUM_GRAPHS)
    xF = P3 + P2 @ p['skip23_W'] + counts[:, None] * p['skip23_b']
    hfc = jax.nn.relu(xF @ p['fc1_W'] + p['fc1_b'])
    return hfc @ p['fc2_W'] + p['fc2_b']


# speedup vs baseline: 5.1300x; 3.5129x over previous
import functools
import jax
import jax.numpy as jnp
from jax import lax
from jax.experimental import pallas as pl
from jax.experimental.pallas import tpu as pltpu
from jax.experimental.pallas import tpu_sc as plsc

N_NODES = 10000
N_EDGES = 160000
NUM_GRAPHS = 256

_VPW = 320
_WIN = 4096
_NPAD = 10240


def _mm_body(a_ref, w_ref, b_ref, o_ref, *, slope):
    acc = jnp.dot(a_ref[...], w_ref[...], preferred_element_type=jnp.float32)
    acc = acc + b_ref[...]
    o_ref[...] = jnp.where(acc >= 0, acc, slope * acc)


def _mm(a, w, b, slope, BM=512, BN=512):
    M, K = a.shape
    _, Nc = w.shape
    Mp = ((M + BM - 1) // BM) * BM
    if Mp != M:
        a = jnp.pad(a, ((0, Mp - M), (0, 0)))
    BNe = min(BN, Nc)
    grid = (Nc // BNe, Mp // BM)
    out = pl.pallas_call(
        functools.partial(_mm_body, slope=slope),
        grid=grid,
        in_specs=[
            pl.BlockSpec((BM, K), lambda j, i: (i, 0)),
            pl.BlockSpec((K, BNe), lambda j, i: (0, j)),
            pl.BlockSpec((1, BNe), lambda j, i: (0, j)),
        ],
        out_specs=pl.BlockSpec((BM, BNe), lambda j, i: (i, j)),
        out_shape=jax.ShapeDtypeStruct((Mp, Nc), jnp.float32),
    )(a, w, b.reshape(1, -1))
    return out[:M] if Mp != M else out


def _make_edge_kernel(C):
    mesh = plsc.VectorSubcoreMesh(core_axis_name="c", subcore_axis_name="s")

    @functools.partial(
        pl.kernel,
        mesh=mesh,
        compiler_params=pltpu.CompilerParams(needs_layout_passes=False),
        out_type=jax.ShapeDtypeStruct((N_NODES, C), jnp.float32),
        scratch_types=[
            pltpu.VMEM((_NPAD,), jnp.float32),
            pltpu.VMEM((_VPW + 16,), jnp.float32),
            pltpu.VMEM((_VPW + 24,), jnp.int32),
            pltpu.VMEM((_WIN,), jnp.int32),
            pltpu.VMEM((16, C), jnp.float32),
            pltpu.VMEM((1, C), jnp.float32),
            pltpu.SemaphoreType.DMA,
        ],
    )
    def edge_kernel(s_hbm, d_hbm, src_hbm, ptr_hbm, h_hbm, out_hbm,
                    s_v, d_v, ptr_v, src_v, rows_v, acc_v, sem):
        wid = lax.axis_index("s") * 2 + lax.axis_index("c")
        v0 = pl.multiple_of(wid * _VPW, 8)
        nv = jnp.minimum(_VPW, N_NODES - v0)

        pltpu.sync_copy(s_hbm, s_v)
        pltpu.sync_copy(d_hbm.at[pl.ds(v0, _VPW + 16)], d_v)
        pltpu.sync_copy(ptr_hbm.at[pl.ds(v0, _VPW + 24)], ptr_v)

        def ensure(e, win_lo):
            need = jnp.logical_or(e < win_lo, e + 16 > win_lo + _WIN)

            def restage():
                new_lo = pl.multiple_of(e - (e % 8), 8)
                pltpu.sync_copy(src_hbm.at[pl.ds(new_lo, _WIN)], src_v)
                return new_lo

            return lax.cond(need, restage, lambda: win_lo)

        def node_body(vl, win_lo):
            pv = ptr_v[pl.ds(vl, 16)]
            e0 = pv[0]
            e1 = pv[1]
            dv = d_v[pl.ds(vl, 16)][0]
            nch = (e1 - e0 + 15) // 16

            def alpha_chunk(e, win_lo):
                idx = src_v[pl.ds(e - win_lo, 16)]
                sg = plsc.load_gather(s_v, [idx])
                t = sg + dv
                a = jnp.where(t >= 0, t, 0.2 * t)
                msk = (lax.iota(jnp.int32, 16) + e) < e1
                return a, msk, idx

            def pass_max(i, carry):
                win_lo, mv = carry
                e = e0 + i * 16
                win_lo = ensure(e, win_lo)
                a, msk, _ = alpha_chunk(e, win_lo)
                mv = jnp.maximum(mv, jnp.where(msk, a, -3.4e38))
                return (win_lo, mv)

            win_lo, mv = lax.fori_loop(
                0, nch, pass_max,
                (win_lo, jnp.full((16,), -3.4e38, jnp.float32)))
            m = jnp.max(mv)

            def pass_den(i, carry):
                win_lo, dacc = carry
                e = e0 + i * 16
                win_lo = ensure(e, win_lo)
                a, msk, _ = alpha_chunk(e, win_lo)
                ea = jnp.exp(a - m)
                dacc = dacc + jnp.where(msk, ea, 0.0)
                return (win_lo, dacc)

            win_lo, dacc = lax.fori_loop(
                0, nch, pass_den, (win_lo, jnp.zeros((16,), jnp.float32)))
            den = jnp.sum(dacc)
            scale = jnp.ones((16,), jnp.float32) / (
                jnp.full((16,), den, jnp.float32) + 1e-16)

            def zero_body(j, _):
                acc_v[0, pl.ds(j * 16, 16)] = jnp.zeros((16,), jnp.float32)
                return 0

            lax.fori_loop(0, C // 16, zero_body, 0)

            def pass_acc(i, win_lo):
                e = e0 + i * 16
                win_lo = ensure(e, win_lo)
                idx = src_v[pl.ds(e - win_lo, 16)]
                cp = pltpu.async_copy(h_hbm.at[idx], rows_v, sem)
                sg = plsc.load_gather(s_v, [idx])
                t = sg + dv
                a = jnp.where(t >= 0, t, 0.2 * t)
                msk = (lax.iota(jnp.int32, 16) + e) < e1
                coef = jnp.where(msk, jnp.exp(a - m) * scale, 0.0)
                cs = [coef[k] for k in range(16)]
                cp.wait()

                def acc_body(j, _):
                    sl = pl.ds(j * 16, 16)
                    ar = acc_v[0, sl]
                    for k in range(16):
                        ar = ar + cs[k] * rows_v[k, sl]
                    acc_v[0, sl] = ar
                    return 0

                lax.fori_loop(0, C // 16, acc_body, 0)
                return win_lo

            win_lo = lax.fori_loop(0, nch, pass_acc, win_lo)
            pltpu.sync_copy(acc_v, out_hbm.at[pl.ds(v0 + vl, 1)])
            return win_lo

        lax.fori_loop(0, nv, node_body, jnp.int32(-2**30))

    return edge_kernel


_edge_kernels = {C: _make_edge_kernel(C) for C in (128, 512, 2048)}


def kernel(x, edge_index, batch, params):
    p = params
    src = edge_index[0]
    dst = edge_index[1]
    eps = 1e-8

    order = jnp.argsort(dst)
    src_s = src[order]
    dst_s = dst[order]
    src_pad = jnp.pad(src_s, (0, _WIN))
    row_ptr = jnp.searchsorted(dst_s, jnp.arange(_NPAD + 64, dtype=jnp.int32),
                               side='left').astype(jnp.int32)

    cy = x[:, 0].astype(jnp.int32)
    cx = x[:, 1].astype(jnp.int32)
    ey = p['coord_emb_y'][cy]
    ex = p['coord_emb_x'][cx]
    coords = jax.nn.leaky_relu(
        jnp.concatenate([ey, ex], 1) @ p['smoosh_W'] + p['smoosh_b'], 0.01)
    pos = x[:, 2].astype(jnp.int32)
    positions = jax.nn.leaky_relu(p['pos_emb'][pos], 0.01)
    ht = x[:, 3:8]
    hold = (ht @ p['hold_W'] + p['hold_b']) / (ht.sum(1, keepdims=True) + eps)
    orr = x[:, 8:16]
    orient = (orr @ p['orient_W'] + p['orient_b']) / (orr.sum(1, keepdims=True) + eps)
    feat = jnp.concatenate([coords, orient, hold, positions], 1)

    def layer(h, Wname, asn, adn, bn):
        W = p[Wname]
        s = jnp.pad(h @ (W @ p[asn]), (0, _NPAD - N_NODES))
        d = jnp.pad(h @ (W @ p[adn]), (0, _NPAD + 64 - N_NODES))
        agg = _edge_kernels[h.shape[1]](s, d, src_pad, row_ptr, h)
        return _mm(agg, W, p[bn], 0.01)

    x1 = layer(feat, 'g1_W', 'g1_as', 'g1_ad', 'g1_b')
    x2 = layer(x1, 'g2_W', 'g2_as', 'g2_ad', 'g2_b')
    x3 = layer(x2, 'g3_W', 'g3_as', 'g3_ad', 'g3_b')

    P3 = jax.ops.segment_sum(x3, batch, num_segments=NUM_GRAPHS)
    P2 = jax.ops.segment_sum(x2, batch, num_segments=NUM_GRAPHS)
    counts = jax.ops.segment_sum(jnp.ones((N_NODES,), jnp.float32), batch,
                                 num_segments=NUM_GRAPHS)
    xF = P3 + P2 @ p['skip23_W'] + counts[:, None] * p['skip23_b']
    hfc = jax.nn.relu(xF @ p['fc1_W'] + p['fc1_b'])
    return hfc @ p['fc2_W'] + p['fc2_b']


# speedup vs baseline: 5.4681x; 1.0659x over previous
import functools
import jax
import jax.numpy as jnp
from jax import lax
from jax.experimental import pallas as pl
from jax.experimental.pallas import tpu as pltpu
from jax.experimental.pallas import tpu_sc as plsc

N_NODES = 10000
N_EDGES = 160000
NUM_GRAPHS = 256

_VPW = 320
_WIN = 4096
_NPAD = 10240


def _mm_body(a_ref, w_ref, b_ref, o_ref, *, slope):
    acc = jnp.dot(a_ref[...], w_ref[...], preferred_element_type=jnp.float32)
    acc = acc + b_ref[...]
    o_ref[...] = jnp.where(acc >= 0, acc, slope * acc)


def _mm(a, w, b, slope, BM=512, BN=512):
    M, K = a.shape
    _, Nc = w.shape
    Mp = ((M + BM - 1) // BM) * BM
    if Mp != M:
        a = jnp.pad(a, ((0, Mp - M), (0, 0)))
    BNe = min(BN, Nc)
    grid = (Nc // BNe, Mp // BM)
    out = pl.pallas_call(
        functools.partial(_mm_body, slope=slope),
        grid=grid,
        in_specs=[
            pl.BlockSpec((BM, K), lambda j, i: (i, 0)),
            pl.BlockSpec((K, BNe), lambda j, i: (0, j)),
            pl.BlockSpec((1, BNe), lambda j, i: (0, j)),
        ],
        out_specs=pl.BlockSpec((BM, BNe), lambda j, i: (i, j)),
        out_shape=jax.ShapeDtypeStruct((Mp, Nc), jnp.float32),
    )(a, w, b.reshape(1, -1))
    return out[:M] if Mp != M else out


def _make_edge_kernel(C):
    mesh = plsc.VectorSubcoreMesh(core_axis_name="c", subcore_axis_name="s")

    @functools.partial(
        pl.kernel,
        mesh=mesh,
        compiler_params=pltpu.CompilerParams(needs_layout_passes=False),
        out_type=jax.ShapeDtypeStruct((N_NODES, C), jnp.float32),
        scratch_types=[
            pltpu.VMEM((_NPAD,), jnp.float32),
            pltpu.VMEM((_VPW + 16,), jnp.float32),
            pltpu.VMEM((_VPW + 24,), jnp.int32),
            pltpu.VMEM((_WIN,), jnp.int32),
            pltpu.VMEM((32, C), jnp.float32),
            pltpu.VMEM((1, C), jnp.float32),
            pltpu.SemaphoreType.DMA,
            pltpu.SemaphoreType.DMA,
        ],
    )
    def edge_kernel(s_hbm, d_hbm, src_hbm, ptr_hbm, h_hbm, out_hbm,
                    s_v, d_v, ptr_v, src_v, rows_v, acc_v, sem, sem2):
        wid = lax.axis_index("s") * 2 + lax.axis_index("c")
        v0 = pl.multiple_of(wid * _VPW, 8)
        nv = jnp.minimum(_VPW, N_NODES - v0)

        pltpu.sync_copy(s_hbm, s_v)
        pltpu.sync_copy(d_hbm.at[pl.ds(v0, _VPW + 16)], d_v)
        pltpu.sync_copy(ptr_hbm.at[pl.ds(v0, _VPW + 24)], ptr_v)

        def ensure(e, win_lo):
            need = jnp.logical_or(e < win_lo, e + 16 > win_lo + _WIN)

            def restage():
                new_lo = pl.multiple_of(e - (e % 8), 8)
                pltpu.sync_copy(src_hbm.at[pl.ds(new_lo, _WIN)], src_v)
                return new_lo

            return lax.cond(need, restage, lambda: win_lo)

        def node_body(vl, win_lo):
            pv = ptr_v[pl.ds(vl, 16)]
            e0 = pv[0]
            e1 = pv[1]
            dv = d_v[pl.ds(vl, 16)][0]
            nch = (e1 - e0 + 15) // 16

            def alpha_chunk(e, win_lo):
                idx = src_v[pl.ds(e - win_lo, 16)]
                sg = plsc.load_gather(s_v, [idx])
                t = sg + dv
                a = jnp.where(t >= 0, t, 0.2 * t)
                msk = (lax.iota(jnp.int32, 16) + e) < e1
                return a, msk, idx

            def pass_max(i, carry):
                win_lo, mv = carry
                e = e0 + i * 16
                win_lo = ensure(e, win_lo)
                a, msk, _ = alpha_chunk(e, win_lo)
                mv = jnp.maximum(mv, jnp.where(msk, a, -3.4e38))
                return (win_lo, mv)

            win_lo, mv = lax.fori_loop(
                0, nch, pass_max,
                (win_lo, jnp.full((16,), -3.4e38, jnp.float32)))
            m = jnp.max(mv)

            def pass_den(i, carry):
                win_lo, dacc = carry
                e = e0 + i * 16
                win_lo = ensure(e, win_lo)
                a, msk, _ = alpha_chunk(e, win_lo)
                ea = jnp.exp(a - m)
                dacc = dacc + jnp.where(msk, ea, 0.0)
                return (win_lo, dacc)

            win_lo, dacc = lax.fori_loop(
                0, nch, pass_den, (win_lo, jnp.zeros((16,), jnp.float32)))
            den = jnp.sum(dacc)
            scale = jnp.ones((16,), jnp.float32) / (
                jnp.full((16,), den, jnp.float32) + 1e-16)

            def zero_body(j, _):
                acc_v[0, pl.ds(j * 16, 16)] = jnp.zeros((16,), jnp.float32)
                return 0

            lax.fori_loop(0, C // 16, zero_body, 0)

            def fire(ci, wl):
                e = e0 + ci * 16
                wl = ensure(e, wl)
                fidx = src_v[pl.ds(e - wl, 16)]
                fpar = lax.rem(ci, 2)

                @pl.when(fpar == 0)
                def _():
                    pltpu.async_copy(h_hbm.at[fidx], rows_v.at[pl.ds(0, 16)],
                                     sem)

                @pl.when(fpar == 1)
                def _():
                    pltpu.async_copy(h_hbm.at[fidx], rows_v.at[pl.ds(16, 16)],
                                     sem2)

                return wl

            win_lo = lax.cond(nch > 0, lambda: fire(0, win_lo),
                              lambda: win_lo)

            def pass_acc(i, win_lo):
                e = e0 + i * 16
                idx = src_v[pl.ds(e - win_lo, 16)]
                sg = plsc.load_gather(s_v, [idx])
                t = sg + dv
                a = jnp.where(t >= 0, t, 0.2 * t)
                msk = (lax.iota(jnp.int32, 16) + e) < e1
                coef = jnp.where(msk, jnp.exp(a - m) * scale, 0.0)
                cs = [coef[k] for k in range(16)]
                win_lo = lax.cond(i + 1 < nch, lambda: fire(i + 1, win_lo),
                                  lambda: win_lo)
                par = lax.rem(i, 2)
                bo = pl.multiple_of(par * 16, 16)

                @pl.when(par == 0)
                def _():
                    pltpu.make_async_copy(h_hbm.at[idx],
                                          rows_v.at[pl.ds(0, 16)], sem).wait()

                @pl.when(par == 1)
                def _():
                    pltpu.make_async_copy(h_hbm.at[idx],
                                          rows_v.at[pl.ds(16, 16)],
                                          sem2).wait()

                def acc_body(j, _):
                    sl = pl.ds(j * 16, 16)
                    ar = acc_v[0, sl]
                    for k in range(16):
                        ar = ar + cs[k] * rows_v[bo + k, sl]
                    acc_v[0, sl] = ar
                    return 0

                lax.fori_loop(0, C // 16, acc_body, 0)
                return win_lo

            win_lo = lax.fori_loop(0, nch, pass_acc, win_lo)
            pltpu.sync_copy(acc_v, out_hbm.at[pl.ds(v0 + vl, 1)])
            return win_lo

        lax.fori_loop(0, nv, node_body, jnp.int32(-2**30))

    return edge_kernel


_edge_kernels = {C: _make_edge_kernel(C) for C in (128, 512, 2048)}


def kernel(x, edge_index, batch, params):
    p = params
    src = edge_index[0]
    dst = edge_index[1]
    eps = 1e-8

    order = jnp.argsort(dst)
    src_s = src[order]
    dst_s = dst[order]
    src_pad = jnp.pad(src_s, (0, _WIN))
    row_ptr = jnp.searchsorted(dst_s, jnp.arange(_NPAD + 64, dtype=jnp.int32),
                               side='left').astype(jnp.int32)

    cy = x[:, 0].astype(jnp.int32)
    cx = x[:, 1].astype(jnp.int32)
    ey = p['coord_emb_y'][cy]
    ex = p['coord_emb_x'][cx]
    coords = jax.nn.leaky_relu(
        jnp.concatenate([ey, ex], 1) @ p['smoosh_W'] + p['smoosh_b'], 0.01)
    pos = x[:, 2].astype(jnp.int32)
    positions = jax.nn.leaky_relu(p['pos_emb'][pos], 0.01)
    ht = x[:, 3:8]
    hold = (ht @ p['hold_W'] + p['hold_b']) / (ht.sum(1, keepdims=True) + eps)
    orr = x[:, 8:16]
    orient = (orr @ p['orient_W'] + p['orient_b']) / (orr.sum(1, keepdims=True) + eps)
    feat = jnp.concatenate([coords, orient, hold, positions], 1)

    def layer(h, Wname, asn, adn, bn):
        W = p[Wname]
        s = jnp.pad(h @ (W @ p[asn]), (0, _NPAD - N_NODES))
        d = jnp.pad(h @ (W @ p[adn]), (0, _NPAD + 64 - N_NODES))
        agg = _edge_kernels[h.shape[1]](s, d, src_pad, row_ptr, h)
        return _mm(agg, W, p[bn], 0.01)

    x1 = layer(feat, 'g1_W', 'g1_as', 'g1_ad', 'g1_b')
    x2 = layer(x1, 'g2_W', 'g2_as', 'g2_ad', 'g2_b')
    x3 = layer(x2, 'g3_W', 'g3_as', 'g3_ad', 'g3_b')

    P3 = jax.ops.segment_sum(x3, batch, num_segments=NUM_GRAPHS)
    P2 = jax.ops.segment_sum(x2, batch, num_segments=NUM_GRAPHS)
    counts = jax.ops.segment_sum(jnp.ones((N_NODES,), jnp.float32), batch,
                                 num_segments=NUM_GRAPHS)
    xF = P3 + P2 @ p['skip23_W'] + counts[:, None] * p['skip23_b']
    hfc = jax.nn.relu(xF @ p['fc1_W'] + p['fc1_b'])
    return hfc @ p['fc2_W'] + p['fc2_b']


# speedup vs baseline: 5.6241x; 1.0285x over previous
import functools
import jax
import jax.numpy as jnp
from jax import lax
from jax.experimental import pallas as pl
from jax.experimental.pallas import tpu as pltpu
from jax.experimental.pallas import tpu_sc as plsc

N_NODES = 10000
N_EDGES = 160000
NUM_GRAPHS = 256

_VPW = 320
_WIN = 4096
_NPAD = 10240


def _mm_body(a_ref, w_ref, b_ref, o_ref, *, slope):
    acc = jnp.dot(a_ref[...], w_ref[...], preferred_element_type=jnp.float32)
    acc = acc + b_ref[...]
    o_ref[...] = jnp.where(acc >= 0, acc, slope * acc)


def _mm(a, w, b, slope, BM=512, BN=512):
    M, K = a.shape
    _, Nc = w.shape
    Mp = ((M + BM - 1) // BM) * BM
    if Mp != M:
        a = jnp.pad(a, ((0, Mp - M), (0, 0)))
    BNe = min(BN, Nc)
    grid = (Nc // BNe, Mp // BM)
    out = pl.pallas_call(
        functools.partial(_mm_body, slope=slope),
        grid=grid,
        in_specs=[
            pl.BlockSpec((BM, K), lambda j, i: (i, 0)),
            pl.BlockSpec((K, BNe), lambda j, i: (0, j)),
            pl.BlockSpec((1, BNe), lambda j, i: (0, j)),
        ],
        out_specs=pl.BlockSpec((BM, BNe), lambda j, i: (i, j)),
        out_shape=jax.ShapeDtypeStruct((Mp, Nc), jnp.float32),
    )(a, w, b.reshape(1, -1))
    return out[:M] if Mp != M else out


def _make_edge_kernel(C):
    mesh = plsc.VectorSubcoreMesh(core_axis_name="c", subcore_axis_name="s")

    @functools.partial(
        pl.kernel,
        mesh=mesh,
        compiler_params=pltpu.CompilerParams(needs_layout_passes=False),
        out_type=jax.ShapeDtypeStruct((N_NODES, C), jnp.float32),
        scratch_types=[
            pltpu.VMEM((_NPAD,), jnp.float32),
            pltpu.VMEM((_VPW + 16,), jnp.float32),
            pltpu.VMEM((_VPW + 24,), jnp.int32),
            pltpu.VMEM((_WIN,), jnp.int32),
            pltpu.VMEM((32, C), jnp.float32),
            pltpu.VMEM((1, C), jnp.float32),
            pltpu.SemaphoreType.DMA,
            pltpu.SemaphoreType.DMA,
        ],
    )
    def edge_kernel(s_hbm, d_hbm, src_hbm, ptr_hbm, h_hbm, out_hbm,
                    s_v, d_v, ptr_v, src_v, rows_v, acc_v, sem, sem2):
        wid = lax.axis_index("s") * 2 + lax.axis_index("c")
        v0 = pl.multiple_of(wid * _VPW, 8)
        nv = jnp.minimum(_VPW, N_NODES - v0)

        pltpu.sync_copy(s_hbm, s_v)
        pltpu.sync_copy(d_hbm.at[pl.ds(v0, _VPW + 16)], d_v)
        pltpu.sync_copy(ptr_hbm.at[pl.ds(v0, _VPW + 24)], ptr_v)

        def ensure(e, win_lo):
            need = jnp.logical_or(e < win_lo, e + 16 > win_lo + _WIN)

            def restage():
                new_lo = pl.multiple_of(e - (e % 8), 8)
                pltpu.sync_copy(src_hbm.at[pl.ds(new_lo, _WIN)], src_v)
                return new_lo

            return lax.cond(need, restage, lambda: win_lo)

        def node_body(vl, win_lo):
            pv = ptr_v[pl.ds(vl, 16)]
            e0 = pv[0]
            e1 = pv[1]
            dv = d_v[pl.ds(vl, 16)][0]
            nch = (e1 - e0 + 15) // 16

            def alpha_chunk(e, win_lo):
                idx = src_v[pl.ds(e - win_lo, 16)]
                sg = plsc.load_gather(s_v, [idx])
                t = sg + dv
                a = jnp.where(t >= 0, t, 0.2 * t)
                msk = (lax.iota(jnp.int32, 16) + e) < e1
                return a, msk, idx

            def pass_max(i, carry):
                win_lo, mv = carry
                e = e0 + i * 16
                win_lo = ensure(e, win_lo)
                a, msk, _ = alpha_chunk(e, win_lo)
                mv = jnp.maximum(mv, jnp.where(msk, a, -3.4e38))
                return (win_lo, mv)

            win_lo, mv = lax.fori_loop(
                0, nch, pass_max,
                (win_lo, jnp.full((16,), -3.4e38, jnp.float32)))
            m = jnp.max(mv)

            def pass_den(i, carry):
                win_lo, dacc = carry
                e = e0 + i * 16
                win_lo = ensure(e, win_lo)
                a, msk, _ = alpha_chunk(e, win_lo)
                ea = jnp.exp(a - m)
                dacc = dacc + jnp.where(msk, ea, 0.0)
                return (win_lo, dacc)

            win_lo, dacc = lax.fori_loop(
                0, nch, pass_den, (win_lo, jnp.zeros((16,), jnp.float32)))
            den = jnp.sum(dacc)
            scale = jnp.ones((16,), jnp.float32) / (
                jnp.full((16,), den, jnp.float32) + 1e-16)

            def zero_body(j, _):
                for u in range(8):
                    acc_v[0, pl.ds(j * 128 + u * 16, 16)] = jnp.zeros(
                        (16,), jnp.float32)
                return 0

            lax.fori_loop(0, C // 128, zero_body, 0)

            def fire(ci, wl):
                e = e0 + ci * 16
                wl = ensure(e, wl)
                fidx = src_v[pl.ds(e - wl, 16)]
                fpar = lax.rem(ci, 2)

                @pl.when(fpar == 0)
                def _():
                    pltpu.async_copy(h_hbm.at[fidx], rows_v.at[pl.ds(0, 16)],
                                     sem)

                @pl.when(fpar == 1)
                def _():
                    pltpu.async_copy(h_hbm.at[fidx], rows_v.at[pl.ds(16, 16)],
                                     sem2)

                return wl

            win_lo = lax.cond(nch > 0, lambda: fire(0, win_lo),
                              lambda: win_lo)

            def pass_acc(i, win_lo):
                e = e0 + i * 16
                idx = src_v[pl.ds(e - win_lo, 16)]
                sg = plsc.load_gather(s_v, [idx])
                t = sg + dv
                a = jnp.where(t >= 0, t, 0.2 * t)
                msk = (lax.iota(jnp.int32, 16) + e) < e1
                coef = jnp.where(msk, jnp.exp(a - m) * scale, 0.0)
                cs = [coef[k] for k in range(16)]
                win_lo = lax.cond(i + 1 < nch, lambda: fire(i + 1, win_lo),
                                  lambda: win_lo)
                par = lax.rem(i, 2)
                bo = pl.multiple_of(par * 16, 16)

                @pl.when(par == 0)
                def _():
                    pltpu.make_async_copy(h_hbm.at[idx],
                                          rows_v.at[pl.ds(0, 16)], sem).wait()

                @pl.when(par == 1)
                def _():
                    pltpu.make_async_copy(h_hbm.at[idx],
                                          rows_v.at[pl.ds(16, 16)],
                                          sem2).wait()

                def acc_body(j, _):
                    for u in range(2):
                        sl = pl.ds(j * 32 + u * 16, 16)
                        ar = acc_v[0, sl]
                        for k in range(16):
                            ar = ar + cs[k] * rows_v[bo + k, sl]
                        acc_v[0, sl] = ar
                    return 0

                lax.fori_loop(0, C // 32, acc_body, 0)
                return win_lo

            win_lo = lax.fori_loop(0, nch, pass_acc, win_lo)
            pltpu.sync_copy(acc_v, out_hbm.at[pl.ds(v0 + vl, 1)])
            return win_lo

        lax.fori_loop(0, nv, node_body, jnp.int32(-2**30))

    return edge_kernel


_edge_kernels = {C: _make_edge_kernel(C) for C in (128, 512, 2048)}


def kernel(x, edge_index, batch, params):
    p = params
    src = edge_index[0]
    dst = edge_index[1]
    eps = 1e-8

    order = jnp.argsort(dst)
    src_s = src[order]
    dst_s = dst[order]
    src_pad = jnp.pad(src_s, (0, _WIN))
    row_ptr = jnp.searchsorted(dst_s, jnp.arange(_NPAD + 64, dtype=jnp.int32),
                               side='left').astype(jnp.int32)

    cy = x[:, 0].astype(jnp.int32)
    cx = x[:, 1].astype(jnp.int32)
    ey = p['coord_emb_y'][cy]
    ex = p['coord_emb_x'][cx]
    coords = jax.nn.leaky_relu(
        jnp.concatenate([ey, ex], 1) @ p['smoosh_W'] + p['smoosh_b'], 0.01)
    pos = x[:, 2].astype(jnp.int32)
    positions = jax.nn.leaky_relu(p['pos_emb'][pos], 0.01)
    ht = x[:, 3:8]
    hold = (ht @ p['hold_W'] + p['hold_b']) / (ht.sum(1, keepdims=True) + eps)
    orr = x[:, 8:16]
    orient = (orr @ p['orient_W'] + p['orient_b']) / (orr.sum(1, keepdims=True) + eps)
    feat = jnp.concatenate([coords, orient, hold, positions], 1)

    def layer(h, Wname, asn, adn, bn):
        W = p[Wname]
        s = jnp.pad(h @ (W @ p[asn]), (0, _NPAD - N_NODES))
        d = jnp.pad(h @ (W @ p[adn]), (0, _NPAD + 64 - N_NODES))
        agg = _edge_kernels[h.shape[1]](s, d, src_pad, row_ptr, h)
        return _mm(agg, W, p[bn], 0.01)

    x1 = layer(feat, 'g1_W', 'g1_as', 'g1_ad', 'g1_b')
    x2 = layer(x1, 'g2_W', 'g2_as', 'g2_ad', 'g2_b')
    x3 = layer(x2, 'g3_W', 'g3_as', 'g3_ad', 'g3_b')

    P3 = jax.ops.segment_sum(x3, batch, num_segments=NUM_GRAPHS)
    P2 = jax.ops.segment_sum(x2, batch, num_segments=NUM_GRAPHS)
    counts = jax.ops.segment_sum(jnp.ones((N_NODES,), jnp.float32), batch,
                                 num_segments=NUM_GRAPHS)
    xF = P3 + P2 @ p['skip23_W'] + counts[:, None] * p['skip23_b']
    hfc = jax.nn.relu(xF @ p['fc1_W'] + p['fc1_b'])
    return hfc @ p['fc2_W'] + p['fc2_b']
